# Initial kernel scaffold; baseline (speedup 1.0000x reference)
import jax
import jax.numpy as jnp
from jax.experimental import pallas as pl


def kernel(x_author, x_paper, emb_W_author, emb_b_author, emb_W_paper, emb_b_paper, l0_writes_Wself, l0_writes_bself, l0_writes_Wneigh, l0_writes_bneigh, l0_wb_Wself, l0_wb_bself, l0_wb_Wneigh, l0_wb_bneigh, l1_writes_Wself, l1_writes_bself, l1_writes_Wneigh, l1_writes_bneigh, l1_wb_Wself, l1_wb_bself, l1_wb_Wneigh, l1_wb_bneigh, ln_author_g, ln_author_b, ln_paper_g, ln_paper_b, out_author_W1, out_author_b1, out_author_W2, out_author_b2, out_paper_W1, out_paper_b1, out_paper_W2, out_paper_b2, edge_writes, edge_wb):
    raise NotImplementedError("write your pallas kernel here")


# speedup vs baseline: 2.1409x; 2.1409x over previous
import functools

import jax
import jax.numpy as jnp
from jax import lax
from jax.experimental import pallas as pl
from jax.experimental.pallas import tpu as pltpu
from jax.experimental.pallas import tpu_sc as plsc

N_AUTHOR = 50000
N_PAPER = 50000
E = 800000
D = 128
HID = 128
OUT = 64

NC = 2
NS = 16
CF = 8
FC = HID // CF
CPC = CF // NC

EB = 128
KI = 16
E_PAD = 819200
ROWS2D = E_PAD // EB
N_OUT = 25
SP_ROWS = 50048
ZROWS = 184
R_PER_S = SP_ROWS // NS

BR = 1000
GRID = N_AUTHOR // BR

_mesh = plsc.VectorSubcoreMesh(core_axis_name="c", subcore_axis_name="s")
_sc_params = pltpu.CompilerParams(use_tc_tiling_on_sc=False)


@functools.partial(
    pl.kernel,
    out_type=[jax.ShapeDtypeStruct((SP_ROWS, FC), jnp.float32)] * CF,
    mesh=_mesh,
    compiler_params=_sc_params,
    scratch_types=[
        pltpu.VMEM((KI, EB), jnp.int32),
        pltpu.VMEM((KI, EB), jnp.int32),
        pltpu.VMEM((KI, EB, FC), jnp.float32),
        pltpu.VMEM((ZROWS, FC), jnp.float32),
        pltpu.VMEM_SHARED((SP_ROWS, FC), jnp.float32),
        pltpu.SemaphoreType.DMA,
        pltpu.SemaphoreType.DMA,
    ],
)
def _sc_segsum(*refs):
    xs = refs[:CF]
    src2d, dst2d = refs[CF], refs[CF + 1]
    os_ = refs[CF + 2:2 * CF + 2]
    src_v, dst_v, rows_v, z_v, acc, gsem, ssem = refs[2 * CF + 2:]

    cid = lax.axis_index("c")
    sid = lax.axis_index("s")

    @pl.loop(0, ZROWS)
    def _zfill(i):
        z_v[i, pl.ds(0, FC)] = jnp.zeros((FC,), jnp.float32)

    def run_chunk(xc, oc):
        @pl.loop(0, R_PER_S // ZROWS)
        def _zero(q):
            pltpu.sync_copy(z_v, acc.at[pl.ds(sid * R_PER_S + q * ZROWS,
                                              ZROWS)])

        plsc.subcore_barrier()

        @pl.loop(0, N_OUT)
        def _outer(o):
            row0 = sid * (N_OUT * KI) + o * KI
            pltpu.sync_copy(src2d.at[pl.ds(row0, KI)], src_v)
            pltpu.sync_copy(dst2d.at[pl.ds(row0, KI)], dst_v)
            hs = [pltpu.async_copy(xc.at[src_v.at[j]], rows_v.at[j], gsem)
                  for j in range(KI)]
            for h in hs:
                h.wait()
            ws = [pltpu.async_copy(rows_v.at[j], acc.at[dst_v.at[j]], ssem,
                                   add=True)
                  for j in range(KI)]
            for h in ws:
                h.wait()

        plsc.subcore_barrier()
        pltpu.sync_copy(acc.at[pl.ds(sid * R_PER_S, R_PER_S)],
                        oc.at[pl.ds(sid * R_PER_S, R_PER_S)])
        plsc.subcore_barrier()

    @pl.when(cid == 0)
    def _():
        for k in range(CPC):
            run_chunk(xs[k], os_[k])

    @pl.when(cid == 1)
    def _():
        for k in range(CPC):
            run_chunk(xs[CPC + k], os_[CPC + k])


@functools.partial(
    pl.kernel,
    out_type=[jax.ShapeDtypeStruct((SP_ROWS, 16), jnp.float32)] * NC,
    mesh=_mesh,
    compiler_params=_sc_params,
    scratch_types=[
        pltpu.VMEM((8, EB), jnp.int32),
        pltpu.VMEM((EB, 16), jnp.float32),
        pltpu.VMEM((ZROWS, 16), jnp.float32),
        pltpu.VMEM_SHARED((SP_ROWS, 16), jnp.float32),
        pltpu.SemaphoreType.DMA,
    ],
)
def _sc_counts(dst2d, c0, c1, dst_v, ones_v, z_v, acc, ssem):
    cid = lax.axis_index("c")
    sid = lax.axis_index("s")

    @pl.loop(0, ZROWS)
    def _zfill(i):
        z_v[i, pl.ds(0, 16)] = jnp.zeros((16,), jnp.float32)

    @pl.loop(0, EB)
    def _ofill(i):
        ones_v[i, pl.ds(0, 16)] = jnp.ones((16,), jnp.float32)

    @pl.loop(0, R_PER_S // ZROWS)
    def _zero(q):
        pltpu.sync_copy(z_v, acc.at[pl.ds(sid * R_PER_S + q * ZROWS, ZROWS)])

    plsc.subcore_barrier()

    def accumulate(base_rows):
        @pl.loop(0, 25)
        def _outer(o):
            row0 = base_rows + sid * 200 + o * 8
            pltpu.sync_copy(dst2d.at[pl.ds(row0, 8)], dst_v)
            ws = [pltpu.async_copy(ones_v, acc.at[dst_v.at[j]], ssem, add=True)
                  for j in range(8)]
            for h in ws:
                h.wait()

    def write_out(cref):
        plsc.subcore_barrier()
        pltpu.sync_copy(acc.at[pl.ds(sid * R_PER_S, R_PER_S)],
                        cref.at[pl.ds(sid * R_PER_S, R_PER_S)])

    @pl.when(cid == 0)
    def _():
        accumulate(0)
        write_out(c0)

    @pl.when(cid == 1)
    def _():
        accumulate(ROWS2D // 2)
        write_out(c1)


def _split(y, outs):
    for k, o in enumerate(outs):
        o[...] = y[:, k * FC:(k + 1) * FC]


def _emb_body(x_ref, w_ref, b_ref, *outs):
    y = jnp.dot(x_ref[...], w_ref[...],
                preferred_element_type=jnp.float32) + b_ref[...]
    _split(y, outs)


def _emb(x, w, b):
    return pl.pallas_call(
        _emb_body,
        grid=(GRID,),
        in_specs=[
            pl.BlockSpec((BR, D), lambda i: (i, 0)),
            pl.BlockSpec((D, HID), lambda i: (0, 0)),
            pl.BlockSpec((1, HID), lambda i: (0, 0)),
        ],
        out_specs=[pl.BlockSpec((BR, FC), lambda i: (i, 0))] * CF,
        out_shape=[jax.ShapeDtypeStruct((N_AUTHOR, FC), jnp.float32)] * CF,
    )(x, w, b)


def _layer_body(*refs):
    xs = refs[:CF]
    ss = refs[CF:2 * CF]
    cnt, ws, bs, wn, bn, g, b = refs[2 * CF:2 * CF + 7]
    outs = refs[2 * CF + 7:]
    x = jnp.concatenate([r[...] for r in xs], axis=-1)
    s = jnp.concatenate([r[...] for r in ss], axis=-1)
    c = cnt[0, :, 0:1] + cnt[1, :, 0:1]
    m = s / jnp.maximum(c, 1.0)
    h = (jnp.dot(x, ws[...], preferred_element_type=jnp.float32)
         + jnp.dot(m, wn[...], preferred_element_type=jnp.float32)
         + bs[...] + bn[...])
    y = jnp.maximum(h, 0.0) + x
    mu = jnp.mean(y, axis=-1, keepdims=True)
    var = jnp.mean((y - mu) ** 2, axis=-1, keepdims=True)
    z = (y - mu) * jax.lax.rsqrt(var + 1e-5) * g[...] + b[...]
    _split(z, outs)


def _layer(xc, sc_, cnt, ws, bs, wn, bn, g, b):
    return pl.pallas_call(
        _layer_body,
        grid=(GRID,),
        in_specs=(
            [pl.BlockSpec((BR, FC), lambda i: (i, 0))] * CF
            + [pl.BlockSpec((BR, FC), lambda i: (i, 0))] * CF
            + [pl.BlockSpec((NC, BR, 16), lambda i: (0, i, 0)),
               pl.BlockSpec((HID, HID), lambda i: (0, 0)),
               pl.BlockSpec((1, HID), lambda i: (0, 0)),
               pl.BlockSpec((HID, HID), lambda i: (0, 0)),
               pl.BlockSpec((1, HID), lambda i: (0, 0)),
               pl.BlockSpec((1, HID), lambda i: (0, 0)),
               pl.BlockSpec((1, HID), lambda i: (0, 0))]
        ),
        out_specs=[pl.BlockSpec((BR, FC), lambda i: (i, 0))] * CF,
        out_shape=[jax.ShapeDtypeStruct((N_AUTHOR, FC), jnp.float32)] * CF,
    )(*xc, *sc_, cnt, ws, bs, wn, bn, g, b)


def _head_body(do_softmax, *refs):
    xs = refs[:CF]
    w1, b1, w2, b2, o_ref = refs[CF:]
    x = jnp.concatenate([r[...] for r in xs], axis=-1)
    h = jnp.maximum(jnp.dot(x, w1[...], preferred_element_type=jnp.float32)
                    + b1[...], 0.0)
    z = jnp.dot(h, w2[...], preferred_element_type=jnp.float32) + b2[...]
    if do_softmax:
        mx = jnp.max(z, axis=-1, keepdims=True)
        lse = jnp.log(jnp.sum(jnp.exp(z - mx), axis=-1, keepdims=True)) + mx
        z = z - lse
    o_ref[...] = z


def _head(xc, w1, b1, w2, b2, do_softmax):
    n_out = w2.shape[1]
    return pl.pallas_call(
        functools.partial(_head_body, do_softmax),
        grid=(GRID,),
        in_specs=(
            [pl.BlockSpec((BR, FC), lambda i: (i, 0))] * CF
            + [pl.BlockSpec((HID, HID), lambda i: (0, 0)),
               pl.BlockSpec((1, HID), lambda i: (0, 0)),
               pl.BlockSpec((HID, n_out), lambda i: (0, 0)),
               pl.BlockSpec((1, n_out), lambda i: (0, 0))]
        ),
        out_specs=pl.BlockSpec((BR, n_out), lambda i: (i, 0)),
        out_shape=jax.ShapeDtypeStruct((N_AUTHOR, n_out), jnp.float32),
    )(*xc, w1, b1, w2, b2)


def _prep_edges(idx, pad_val):
    idx = idx.astype(jnp.int32)
    pad = jnp.full((E_PAD - E,), pad_val, jnp.int32)
    return jnp.concatenate([idx, pad]).reshape(ROWS2D, EB)


def kernel(x_author, x_paper, emb_W_author, emb_b_author, emb_W_paper,
           emb_b_paper, l0_writes_Wself, l0_writes_bself, l0_writes_Wneigh,
           l0_writes_bneigh, l0_wb_Wself, l0_wb_bself, l0_wb_Wneigh,
           l0_wb_bneigh, l1_writes_Wself, l1_writes_bself, l1_writes_Wneigh,
           l1_writes_bneigh, l1_wb_Wself, l1_wb_bself, l1_wb_Wneigh,
           l1_wb_bneigh, ln_author_g, ln_author_b, ln_paper_g, ln_paper_b,
           out_author_W1, out_author_b1, out_author_W2, out_author_b2,
           out_paper_W1, out_paper_b1, out_paper_W2, out_paper_b2,
           edge_writes, edge_wb):
    r = lambda v: v.reshape(1, -1)

    src_w = _prep_edges(edge_writes[0], 0)
    dst_w = _prep_edges(edge_writes[1], N_PAPER)
    src_b = _prep_edges(edge_wb[0], 0)
    dst_b = _prep_edges(edge_wb[1], N_AUTHOR)

    cnt_w = jnp.stack(_sc_counts(dst_w))
    cnt_b = jnp.stack(_sc_counts(dst_b))

    xa = _emb(x_author, emb_W_author, r(emb_b_author))
    xp = _emb(x_paper, emb_W_paper, r(emb_b_paper))

    for (ws_w, bs_w, wn_w, bn_w, ws_b, bs_b, wn_b, bn_b) in (
        (l0_writes_Wself, l0_writes_bself, l0_writes_Wneigh, l0_writes_bneigh,
         l0_wb_Wself, l0_wb_bself, l0_wb_Wneigh, l0_wb_bneigh),
        (l1_writes_Wself, l1_writes_bself, l1_writes_Wneigh, l1_writes_bneigh,
         l1_wb_Wself, l1_wb_bself, l1_wb_Wneigh, l1_wb_bneigh),
    ):
        m_p = _sc_segsum(*xa, src_w, dst_w)
        m_a = _sc_segsum(*xp, src_b, dst_b)
        xp = _layer(xp, m_p, cnt_w, ws_w, r(bs_w), wn_w, r(bn_w),
                    r(ln_paper_g), r(ln_paper_b))
        xa = _layer(xa, m_a, cnt_b, ws_b, r(bs_b), wn_b, r(bn_b),
                    r(ln_author_g), r(ln_author_b))

    out_a = _head(xa, out_author_W1, r(out_author_b1), out_author_W2,
                  r(out_author_b2), True)
    out_p = _head(xp, out_paper_W1, r(out_paper_b1), out_paper_W2,
                  r(out_paper_b2), False)
    return out_a, out_p


# speedup vs baseline: 4.4667x; 2.0864x over previous
import functools

import jax
import jax.numpy as jnp
from jax import lax
from jax.experimental import pallas as pl
from jax.experimental.pallas import tpu as pltpu
from jax.experimental.pallas import tpu_sc as plsc

N_AUTHOR = 50000
N_PAPER = 50000
E = 800000
D = 128
HID = 128
OUT = 64

NC = 2
NS = 16
CF = 4
FC = HID // CF
CPC = CF // NC

EB = 128
GI = 25
E_PAD = 819200
ROWS2D = E_PAD // EB
N_OUT = 16
SP_ROWS = 50048
ZROWS = 184
R_PER_S = SP_ROWS // NS

BR = 1000
GRID = N_AUTHOR // BR

_mesh = plsc.VectorSubcoreMesh(core_axis_name="c", subcore_axis_name="s")
_sc_params = pltpu.CompilerParams(use_tc_tiling_on_sc=False)


@functools.partial(
    pl.kernel,
    out_type=[jax.ShapeDtypeStruct((SP_ROWS, FC), jnp.bfloat16)] * CF,
    mesh=_mesh,
    compiler_params=_sc_params,
    scratch_types=[
        pltpu.VMEM((2 * GI, EB), jnp.int32),
        pltpu.VMEM((2 * GI, EB), jnp.int32),
        pltpu.VMEM((GI, EB, FC), jnp.bfloat16),
        pltpu.VMEM((ZROWS, FC), jnp.bfloat16),
        pltpu.VMEM_SHARED((SP_ROWS, FC), jnp.bfloat16),
        pltpu.SemaphoreType.DMA,
        pltpu.SemaphoreType.DMA,
        pltpu.SemaphoreType.DMA,
    ],
)
def _sc_segsum(*refs):
    xs = refs[:CF]
    src2d, dst2d = refs[CF], refs[CF + 1]
    os_ = refs[CF + 2:2 * CF + 2]
    src_v, dst_v, rows_v, z_v, acc, gsem, ssem, isem = refs[2 * CF + 2:]

    cid = lax.axis_index("c")
    sid = lax.axis_index("s")

    @pl.loop(0, ZROWS)
    def _zfill(i):
        z_v[i, pl.ds(0, FC)] = jnp.zeros((FC,), jnp.bfloat16)

    def idx_rows(o):
        return sid * (N_OUT * GI) + o * GI

    def run_chunk(xc, oc):
        def fire_idx(o):
            b = lax.rem(o, 2) * GI
            pltpu.async_copy(src2d.at[pl.ds(idx_rows(o), GI)],
                             src_v.at[pl.ds(b, GI)], isem)
            pltpu.async_copy(dst2d.at[pl.ds(idx_rows(o), GI)],
                             dst_v.at[pl.ds(b, GI)], isem)

        @pl.loop(0, R_PER_S // ZROWS)
        def _zero(q):
            pltpu.sync_copy(z_v, acc.at[pl.ds(sid * R_PER_S + q * ZROWS,
                                              ZROWS)])

        plsc.subcore_barrier()

        def wait_idx(o):
            b = lax.rem(o, 2) * GI
            pltpu.make_async_copy(src2d.at[pl.ds(idx_rows(o), GI)],
                                  src_v.at[pl.ds(b, GI)], isem).wait()
            pltpu.make_async_copy(dst2d.at[pl.ds(idx_rows(o), GI)],
                                  dst_v.at[pl.ds(b, GI)], isem).wait()

        pltpu.sync_copy(src2d.at[pl.ds(idx_rows(0), GI)],
                        src_v.at[pl.ds(0, GI)])
        pltpu.sync_copy(dst2d.at[pl.ds(idx_rows(0), GI)],
                        dst_v.at[pl.ds(0, GI)])
        fire_idx(1)

        @pl.loop(0, N_OUT)
        def _outer(o):
            b = lax.rem(o, 2) * GI
            hg = [pltpu.async_copy(xc.at[src_v.at[b + j]], rows_v.at[j],
                                   gsem)
                  for j in range(GI)]
            hs = []
            for j in range(GI):
                hg[j].wait()
                hs.append(pltpu.async_copy(rows_v.at[j],
                                           acc.at[dst_v.at[b + j]], ssem,
                                           add=True))
            for h in hs:
                h.wait()

            @pl.when(o < N_OUT - 1)
            def _():
                wait_idx(o + 1)

            @pl.when(o < N_OUT - 2)
            def _():
                fire_idx(o + 2)

        plsc.subcore_barrier()
        pltpu.sync_copy(acc.at[pl.ds(sid * R_PER_S, R_PER_S)],
                        oc.at[pl.ds(sid * R_PER_S, R_PER_S)])
        plsc.subcore_barrier()

    @pl.when(cid == 0)
    def _():
        for k in range(CPC):
            run_chunk(xs[k], os_[k])

    @pl.when(cid == 1)
    def _():
        for k in range(CPC):
            run_chunk(xs[CPC + k], os_[CPC + k])


@functools.partial(
    pl.kernel,
    out_type=[jax.ShapeDtypeStruct((SP_ROWS, 16), jnp.float32)] * NC,
    mesh=_mesh,
    compiler_params=_sc_params,
    scratch_types=[
        pltpu.VMEM((8, EB), jnp.int32),
        pltpu.VMEM((EB, 16), jnp.float32),
        pltpu.VMEM((ZROWS, 16), jnp.float32),
        pltpu.VMEM_SHARED((SP_ROWS, 16), jnp.float32),
        pltpu.SemaphoreType.DMA,
    ],
)
def _sc_counts(dst_w2, dst_b2, c0, c1, dst_v, ones_v, z_v, acc, ssem):
    cid = lax.axis_index("c")
    sid = lax.axis_index("s")

    @pl.loop(0, ZROWS)
    def _zfill(i):
        z_v[i, pl.ds(0, 16)] = jnp.zeros((16,), jnp.float32)

    @pl.loop(0, EB)
    def _ofill(i):
        ones_v[i, pl.ds(0, 16)] = jnp.ones((16,), jnp.float32)

    @pl.loop(0, R_PER_S // ZROWS)
    def _zero(q):
        pltpu.sync_copy(z_v, acc.at[pl.ds(sid * R_PER_S + q * ZROWS, ZROWS)])

    plsc.subcore_barrier()

    def accumulate(dst2d):
        @pl.loop(0, 50)
        def _outer(o):
            row0 = sid * 400 + o * 8
            pltpu.sync_copy(dst2d.at[pl.ds(row0, 8)], dst_v)
            ws = [pltpu.async_copy(ones_v, acc.at[dst_v.at[j]], ssem,
                                   add=True)
                  for j in range(8)]
            for h in ws:
                h.wait()

    def write_out(cref):
        plsc.subcore_barrier()
        pltpu.sync_copy(acc.at[pl.ds(sid * R_PER_S, R_PER_S)],
                        cref.at[pl.ds(sid * R_PER_S, R_PER_S)])

    @pl.when(cid == 0)
    def _():
        accumulate(dst_w2)
        write_out(c0)

    @pl.when(cid == 1)
    def _():
        accumulate(dst_b2)
        write_out(c1)


def _split(y, outs):
    f32s, b16s = outs[:CF], outs[CF:]
    for k in range(CF):
        blk = y[:, k * FC:(k + 1) * FC]
        f32s[k][...] = blk
        b16s[k][...] = blk.astype(jnp.bfloat16)


def _emb_body(x_ref, w_ref, b_ref, *outs):
    y = jnp.dot(x_ref[...], w_ref[...],
                preferred_element_type=jnp.float32) + b_ref[...]
    _split(y, outs)


def _emb(x, w, b):
    return pl.pallas_call(
        _emb_body,
        grid=(GRID,),
        in_specs=[
            pl.BlockSpec((BR, D), lambda i: (i, 0)),
            pl.BlockSpec((D, HID), lambda i: (0, 0)),
            pl.BlockSpec((1, HID), lambda i: (0, 0)),
        ],
        out_specs=[pl.BlockSpec((BR, FC), lambda i: (i, 0))] * (2 * CF),
        out_shape=([jax.ShapeDtypeStruct((N_AUTHOR, FC), jnp.float32)] * CF
                   + [jax.ShapeDtypeStruct((N_AUTHOR, FC), jnp.bfloat16)]
                   * CF),
    )(x, w, b)


def _one_layer(xs, ss, cnt, ws, bs, wn, bn, g, b):
    x = jnp.concatenate([r[...] for r in xs], axis=-1)
    s = jnp.concatenate([r[...].astype(jnp.float32) for r in ss], axis=-1)
    c = cnt[:, 0:1]
    m = s / jnp.maximum(c, 1.0)
    h = (jnp.dot(x, ws[...], preferred_element_type=jnp.float32)
         + jnp.dot(m, wn[...], preferred_element_type=jnp.float32)
         + bs[...] + bn[...])
    y = jnp.maximum(h, 0.0) + x
    mu = jnp.mean(y, axis=-1, keepdims=True)
    var = jnp.mean((y - mu) ** 2, axis=-1, keepdims=True)
    return (y - mu) * jax.lax.rsqrt(var + 1e-5) * g[...] + b[...]


def _layer_body(*refs):
    xs = refs[:CF]
    ss = refs[CF:2 * CF]
    outs = refs[2 * CF + 7:]
    z = _one_layer(xs, ss, *refs[2 * CF:2 * CF + 7])
    _split(z, outs)


def _layer(xc, sc_, cnt, ws, bs, wn, bn, g, b):
    return pl.pallas_call(
        _layer_body,
        grid=(GRID,),
        in_specs=(
            [pl.BlockSpec((BR, FC), lambda i: (i, 0))] * CF
            + [pl.BlockSpec((BR, FC), lambda i: (i, 0))] * CF
            + [pl.BlockSpec((BR, 16), lambda i: (i, 0)),
               pl.BlockSpec((HID, HID), lambda i: (0, 0)),
               pl.BlockSpec((1, HID), lambda i: (0, 0)),
               pl.BlockSpec((HID, HID), lambda i: (0, 0)),
               pl.BlockSpec((1, HID), lambda i: (0, 0)),
               pl.BlockSpec((1, HID), lambda i: (0, 0)),
               pl.BlockSpec((1, HID), lambda i: (0, 0))]
        ),
        out_specs=[pl.BlockSpec((BR, FC), lambda i: (i, 0))] * (2 * CF),
        out_shape=([jax.ShapeDtypeStruct((N_AUTHOR, FC), jnp.float32)] * CF
                   + [jax.ShapeDtypeStruct((N_AUTHOR, FC), jnp.bfloat16)]
                   * CF),
    )(*xc, *sc_, cnt, ws, bs, wn, bn, g, b)


def _head_body(do_softmax, *refs):
    xs = refs[:CF]
    w1, b1, w2, b2, o_ref = refs[CF:]
    x = jnp.concatenate([r[...] for r in xs], axis=-1)
    h = jnp.maximum(jnp.dot(x, w1[...], preferred_element_type=jnp.float32)
                    + b1[...], 0.0)
    z = jnp.dot(h, w2[...], preferred_element_type=jnp.float32) + b2[...]
    if do_softmax:
        mx = jnp.max(z, axis=-1, keepdims=True)
        lse = jnp.log(jnp.sum(jnp.exp(z - mx), axis=-1, keepdims=True)) + mx
        z = z - lse
    o_ref[...] = z


def _head(xc, w1, b1, w2, b2, do_softmax):
    n_out = w2.shape[1]
    return pl.pallas_call(
        functools.partial(_head_body, do_softmax),
        grid=(GRID,),
        in_specs=(
            [pl.BlockSpec((BR, FC), lambda i: (i, 0))] * CF
            + [pl.BlockSpec((HID, HID), lambda i: (0, 0)),
               pl.BlockSpec((1, HID), lambda i: (0, 0)),
               pl.BlockSpec((HID, n_out), lambda i: (0, 0)),
               pl.BlockSpec((1, n_out), lambda i: (0, 0))]
        ),
        out_specs=pl.BlockSpec((BR, n_out), lambda i: (i, 0)),
        out_shape=jax.ShapeDtypeStruct((N_AUTHOR, n_out), jnp.float32),
    )(*xc, w1, b1, w2, b2)


def _prep_edges(idx, pad_val):
    idx = idx.astype(jnp.int32)
    pad = jnp.full((E_PAD - E,), pad_val, jnp.int32)
    return jnp.concatenate([idx, pad]).reshape(ROWS2D, EB)


def kernel(x_author, x_paper, emb_W_author, emb_b_author, emb_W_paper,
           emb_b_paper, l0_writes_Wself, l0_writes_bself, l0_writes_Wneigh,
           l0_writes_bneigh, l0_wb_Wself, l0_wb_bself, l0_wb_Wneigh,
           l0_wb_bneigh, l1_writes_Wself, l1_writes_bself, l1_writes_Wneigh,
           l1_writes_bneigh, l1_wb_Wself, l1_wb_bself, l1_wb_Wneigh,
           l1_wb_bneigh, ln_author_g, ln_author_b, ln_paper_g, ln_paper_b,
           out_author_W1, out_author_b1, out_author_W2, out_author_b2,
           out_paper_W1, out_paper_b1, out_paper_W2, out_paper_b2,
           edge_writes, edge_wb):
    r = lambda v: v.reshape(1, -1)

    src_w = _prep_edges(edge_writes[0], 0)
    dst_w = _prep_edges(edge_writes[1], N_PAPER)
    src_b = _prep_edges(edge_wb[0], 0)
    dst_b = _prep_edges(edge_wb[1], N_AUTHOR)

    cnt_w, cnt_b = _sc_counts(dst_w, dst_b)

    xa_all = _emb(x_author, emb_W_author, r(emb_b_author))
    xp_all = _emb(x_paper, emb_W_paper, r(emb_b_paper))
    xa, xa16 = xa_all[:CF], xa_all[CF:]
    xp, xp16 = xp_all[:CF], xp_all[CF:]

    lp = dict(ws=l0_writes_Wself, bs=r(l0_writes_bself), wn=l0_writes_Wneigh,
              bn=r(l0_writes_bneigh), g=r(ln_paper_g), b=r(ln_paper_b))
    la = dict(ws=l0_wb_Wself, bs=r(l0_wb_bself), wn=l0_wb_Wneigh,
              bn=r(l0_wb_bneigh), g=r(ln_author_g), b=r(ln_author_b))
    lp1 = dict(ws=l1_writes_Wself, bs=r(l1_writes_bself), wn=l1_writes_Wneigh,
               bn=r(l1_writes_bneigh), g=r(ln_paper_g), b=r(ln_paper_b))
    la1 = dict(ws=l1_wb_Wself, bs=r(l1_wb_bself), wn=l1_wb_Wneigh,
               bn=r(l1_wb_bneigh), g=r(ln_author_g), b=r(ln_author_b))

    def lay(xc, m, cnt, w):
        out = _layer(xc, m, cnt, w["ws"], w["bs"], w["wn"], w["bn"],
                     w["g"], w["b"])
        return out[:CF], out[CF:]

    m_p0 = _sc_segsum(*xa16, src_w, dst_w)
    m_a0 = _sc_segsum(*xp16, src_b, dst_b)
    xp, xp16 = lay(xp, m_p0, cnt_w, lp)
    m_a1 = _sc_segsum(*xp16, src_b, dst_b)
    xa, xa16 = lay(xa, m_a0, cnt_b, la)
    m_p1 = _sc_segsum(*xa16, src_w, dst_w)
    xa, xa16 = lay(xa, m_a1, cnt_b, la1)
    xp, xp16 = lay(xp, m_p1, cnt_w, lp1)

    out_a = _head(xa, out_author_W1, r(out_author_b1), out_author_W2,
                  r(out_author_b2), True)
    out_p = _head(xp, out_paper_W1, r(out_paper_b1), out_paper_W2,
                  r(out_paper_b2), False)
    return out_a, out_p
